# Initial kernel scaffold; baseline (speedup 1.0000x reference)
import functools
import jax
import jax.numpy as jnp
from jax import lax
from jax.experimental import pallas as pl
from jax.experimental.pallas import tpu as pltpu
from jax.experimental.pallas import tpu_sc as plsc

N = 10000
E = 320000
B = 64
H = 256
NHALF = 5000
NPAD = 5008
NW = 32
EW = E // NW
ET = E // 16
KC = 80

_MESH = plsc.VectorSubcoreMesh(core_axis_name="c", subcore_axis_name="s")


def _lrelu(v):
    return jnp.where(v >= 0, v, 0.01 * v)


def _dot_t(a, b):
    return lax.dot_general(a, b, (((1,), (1,)), ((), ())),
                           preferred_element_type=jnp.float32)


def _tc_a_body(x_ref, w1_ref, b1_ref, wl_ref, w2_ref, ar_ref,
               xe_ref, xw1_ref, xw2_ref, r_ref):
    xe = _lrelu(_dot_t(x_ref[...], w1_ref[...]) + b1_ref[...])
    xe_ref[...] = xe
    xw1_ref[...] = _dot_t(xe, wl_ref[...])
    xw2_ref[...] = _dot_t(xe, w2_ref[...])
    r_ref[...] = jnp.dot(xe, ar_ref[...], preferred_element_type=jnp.float32)


def _tc_a(x, w1, b1, wl, w2, ar):
    blk = 500
    f = pl.pallas_call(
        _tc_a_body,
        grid=(N // blk,),
        in_specs=[
            pl.BlockSpec((blk, 128), lambda i: (i, 0)),
            pl.BlockSpec((H, 128), lambda i: (0, 0)),
            pl.BlockSpec((1, H), lambda i: (0, 0)),
            pl.BlockSpec((H, H), lambda i: (0, 0)),
            pl.BlockSpec((H, H), lambda i: (0, 0)),
            pl.BlockSpec((H, 1), lambda i: (0, 0)),
        ],
        out_specs=[
            pl.BlockSpec((blk, H), lambda i: (i, 0)),
            pl.BlockSpec((blk, H), lambda i: (i, 0)),
            pl.BlockSpec((blk, H), lambda i: (i, 0)),
            pl.BlockSpec((blk, 1), lambda i: (i, 0)),
        ],
        out_shape=[
            jax.ShapeDtypeStruct((N, H), jnp.float32),
            jax.ShapeDtypeStruct((N, H), jnp.float32),
            jax.ShapeDtypeStruct((N, H), jnp.float32),
            jax.ShapeDtypeStruct((N, 1), jnp.float32),
        ],
    )
    return f(x, w1, b1.reshape(1, H), wl, w2, ar.reshape(H, 1))


def _tc_eaw_body(ea_ref, wrt_ref, out_ref):
    out_ref[...] = jnp.dot(ea_ref[...], wrt_ref[...],
                           preferred_element_type=jnp.float32)


def _tc_eaw(ea, wr):
    blk = 1600
    f = pl.pallas_call(
        _tc_eaw_body,
        grid=(E // blk,),
        in_specs=[
            pl.BlockSpec((blk, 16), lambda i: (i, 0)),
            pl.BlockSpec((16, H), lambda i: (0, 0)),
        ],
        out_specs=pl.BlockSpec((blk, H), lambda i: (i, 0)),
        out_shape=jax.ShapeDtypeStruct((E, H), jnp.float32),
    )
    return f(ea, wr.T)


def _tc_layer_body(rows_ref, d_ref, ba_ref, x_ref, wih_ref, whh_ref,
                   bih_ref, bhh_ref, wn_ref, ans_ref, and_ref,
                   xn_ref, xl_ref, as_ref, ad_ref):
    h = rows_ref[...] / (d_ref[...] + 1e-16) + ba_ref[...]
    h = jnp.where(h > 0, h, jnp.expm1(h))
    xold = x_ref[...]
    gi = _dot_t(h, wih_ref[...]) + bih_ref[...]
    gh = _dot_t(xold, whh_ref[...]) + bhh_ref[...]
    r = jax.nn.sigmoid(gi[:, :H] + gh[:, :H])
    z = jax.nn.sigmoid(gi[:, H:2 * H] + gh[:, H:2 * H])
    n = jnp.tanh(gi[:, 2 * H:] + r * gh[:, 2 * H:])
    xn = jnp.maximum((1.0 - z) * n + z * xold, 0.0)
    xn_ref[...] = xn
    xl = _dot_t(xn, wn_ref[...])
    xl_ref[...] = xl
    as_ref[...] = jnp.dot(xl, ans_ref[...], preferred_element_type=jnp.float32)
    ad_ref[...] = jnp.dot(xl, and_ref[...], preferred_element_type=jnp.float32)


def _tc_layer(rows, d, ba, x, wih, whh, bih, bhh, wn, ans, andst):
    blk = 500
    f = pl.pallas_call(
        _tc_layer_body,
        grid=(N // blk,),
        in_specs=[
            pl.BlockSpec((blk, H), lambda i: (i, 0)),
            pl.BlockSpec((blk, 1), lambda i: (i, 0)),
            pl.BlockSpec((1, H), lambda i: (0, 0)),
            pl.BlockSpec((blk, H), lambda i: (i, 0)),
            pl.BlockSpec((3 * H, H), lambda i: (0, 0)),
            pl.BlockSpec((3 * H, H), lambda i: (0, 0)),
            pl.BlockSpec((1, 3 * H), lambda i: (0, 0)),
            pl.BlockSpec((1, 3 * H), lambda i: (0, 0)),
            pl.BlockSpec((H, H), lambda i: (0, 0)),
            pl.BlockSpec((H, 1), lambda i: (0, 0)),
            pl.BlockSpec((H, 1), lambda i: (0, 0)),
        ],
        out_specs=[
            pl.BlockSpec((blk, H), lambda i: (i, 0)),
            pl.BlockSpec((blk, H), lambda i: (i, 0)),
            pl.BlockSpec((blk, 1), lambda i: (i, 0)),
            pl.BlockSpec((blk, 1), lambda i: (i, 0)),
        ],
        out_shape=[
            jax.ShapeDtypeStruct((N, H), jnp.float32),
            jax.ShapeDtypeStruct((N, H), jnp.float32),
            jax.ShapeDtypeStruct((N, 1), jnp.float32),
            jax.ShapeDtypeStruct((N, 1), jnp.float32),
        ],
    )
    return f(rows, d, ba.reshape(1, H), x, wih, whh, bih.reshape(1, 3 * H),
             bhh.reshape(1, 3 * H), wn, ans.reshape(H, 1), andst.reshape(H, 1))


def _mask_t(batch_blk):
    blk = batch_blk.shape[0]
    cols = lax.broadcasted_iota(jnp.int32, (blk, B), 1)
    return jnp.where(batch_blk == cols, 1.0, 0.0).astype(jnp.float32)


def _dot_c0(a, b):
    return lax.dot_general(a, b, (((0,), (0,)), ((), ())),
                           preferred_element_type=jnp.float32)


def _tc_r1_body(x_ref, b_ref, molw_ref, ad_ref, out_ref, cd_ref, acc):
    i = pl.program_id(0)

    @pl.when(i == 0)
    def _():
        acc[...] = jnp.zeros_like(acc)

    mt = _mask_t(b_ref[...])
    acc[...] += _dot_c0(mt, x_ref[...])

    @pl.when(i == pl.num_programs(0) - 1)
    def _():
        out0 = jnp.maximum(acc[...], 0.0)
        out_ref[...] = out0
        od = _dot_t(out0, molw_ref[...])
        cd_ref[...] = jnp.dot(od, ad_ref[...], preferred_element_type=jnp.float32)


def _tc_r1(x, batch, molw, attdst):
    blk = 500
    f = pl.pallas_call(
        _tc_r1_body,
        grid=(N // blk,),
        in_specs=[
            pl.BlockSpec((blk, H), lambda i: (i, 0)),
            pl.BlockSpec((blk, 1), lambda i: (i, 0)),
            pl.BlockSpec((H, H), lambda i: (0, 0)),
            pl.BlockSpec((H, 1), lambda i: (0, 0)),
        ],
        out_specs=[
            pl.BlockSpec((B, H), lambda i: (0, 0)),
            pl.BlockSpec((B, 1), lambda i: (0, 0)),
        ],
        out_shape=[
            jax.ShapeDtypeStruct((B, H), jnp.float32),
            jax.ShapeDtypeStruct((B, 1), jnp.float32),
        ],
        scratch_shapes=[pltpu.VMEM((B, H), jnp.float32)],
    )
    return f(x, batch, molw, attdst.reshape(H, 1))


def _tc_r3_body(cs_ref, cd_ref, b_ref, s_ref, g_ref, gacc):
    i = pl.program_id(0)
    mt = _mask_t(b_ref[...])
    s = _lrelu(cs_ref[...] + jnp.dot(mt, cd_ref[...],
                                     preferred_element_type=jnp.float32))
    s_ref[...] = s
    m = jnp.max(s)

    @pl.when(i == 0)
    def _():
        gacc[0, 0] = m

    @pl.when(i > 0)
    def _():
        gacc[0, 0] = jnp.maximum(gacc[0, 0], m)

    @pl.when(i == pl.num_programs(0) - 1)
    def _():
        g_ref[0, 0] = gacc[0, 0]


def _tc_r3(csrc, cdst, batch):
    blk = 500
    f = pl.pallas_call(
        _tc_r3_body,
        grid=(N // blk,),
        in_specs=[
            pl.BlockSpec((blk, 1), lambda i: (i, 0)),
            pl.BlockSpec((B, 1), lambda i: (0, 0)),
            pl.BlockSpec((blk, 1), lambda i: (i, 0)),
        ],
        out_specs=[
            pl.BlockSpec((blk, 1), lambda i: (i, 0)),
            pl.BlockSpec((1, 1), lambda i: (0, 0)),
        ],
        out_shape=[
            jax.ShapeDtypeStruct((N, 1), jnp.float32),
            jax.ShapeDtypeStruct((1, 1), jnp.float32),
        ],
        scratch_shapes=[pltpu.SMEM((1, 1), jnp.float32)],
    )
    return f(csrc, cdst, batch)


def _tc_r4_body(xs_ref, s_ref, g_ref, b_ref, bias_ref, h_ref, dacc, hacc):
    i = pl.program_id(0)

    @pl.when(i == 0)
    def _():
        dacc[...] = jnp.zeros_like(dacc)
        hacc[...] = jnp.zeros_like(hacc)

    mt = _mask_t(b_ref[...])
    e = jnp.exp(s_ref[...] - g_ref[...])
    dacc[...] += _dot_c0(mt, e)
    hacc[...] += _dot_c0(mt, xs_ref[...] * e)

    @pl.when(i == pl.num_programs(0) - 1)
    def _():
        h = hacc[...] / (dacc[...] + 1e-16) + bias_ref[...]
        h_ref[...] = jnp.where(h > 0, h, jnp.expm1(h))


def _tc_r4(xs, s, g, batch, bias):
    blk = 500
    f = pl.pallas_call(
        _tc_r4_body,
        grid=(N // blk,),
        in_specs=[
            pl.BlockSpec((blk, H), lambda i: (i, 0)),
            pl.BlockSpec((blk, 1), lambda i: (i, 0)),
            pl.BlockSpec((1, 1), lambda i: (0, 0)),
            pl.BlockSpec((blk, 1), lambda i: (i, 0)),
            pl.BlockSpec((1, H), lambda i: (0, 0)),
        ],
        out_specs=pl.BlockSpec((B, H), lambda i: (0, 0)),
        out_shape=jax.ShapeDtypeStruct((B, H), jnp.float32),
        scratch_shapes=[pltpu.VMEM((B, 1), jnp.float32),
                        pltpu.VMEM((B, H), jnp.float32)],
    )
    return f(xs, s, g, batch, bias.reshape(1, H))


def _tc_r5_body(h_ref, o_ref, wih_ref, whh_ref, bih_ref, bhh_ref,
                molw_ref, ad_ref, on_ref, cd_ref):
    h = h_ref[...]
    oo = o_ref[...]
    gi = _dot_t(h, wih_ref[...]) + bih_ref[...]
    gh = _dot_t(oo, whh_ref[...]) + bhh_ref[...]
    r = jax.nn.sigmoid(gi[:, :H] + gh[:, :H])
    z = jax.nn.sigmoid(gi[:, H:2 * H] + gh[:, H:2 * H])
    n = jnp.tanh(gi[:, 2 * H:] + r * gh[:, 2 * H:])
    on = jnp.maximum((1.0 - z) * n + z * oo, 0.0)
    on_ref[...] = on
    od = _dot_t(on, molw_ref[...])
    cd_ref[...] = jnp.dot(od, ad_ref[...], preferred_element_type=jnp.float32)


def _tc_r5(h, out, wih, whh, bih, bhh, molw, attdst):
    f = pl.pallas_call(
        _tc_r5_body,
        out_shape=[
            jax.ShapeDtypeStruct((B, H), jnp.float32),
            jax.ShapeDtypeStruct((B, 1), jnp.float32),
        ],
    )
    return f(h, out, wih, whh, bih.reshape(1, 3 * H), bhh.reshape(1, 3 * H),
             molw, attdst.reshape(H, 1))


def _tc_r6_body(o_ref, w_ref, b_ref, out_ref):
    out_ref[...] = _dot_t(o_ref[...], w_ref[...]) + b_ref[...]


def _tc_r6(out, w, b):
    f = pl.pallas_call(
        _tc_r6_body,
        out_shape=jax.ShapeDtypeStruct((B, w.shape[0]), jnp.float32),
    )
    return f(out, w, b.reshape(1, -1))


def _widx():
    return lax.axis_index("c") * 16 + lax.axis_index("s")


def _sc_score_e_kernel(eaw_hbm, xw1_hbm, src_hbm, dst_hbm, rr_hbm, attl_hbm,
                       s_hbm, mx_hbm,
                       rbuf, albuf, srcb, dstb, eab, g1b, pbuf, sbuf, mbuf,
                       sem):
    wid = _widx()
    base = wid * EW
    pltpu.sync_copy(rr_hbm, rbuf)
    pltpu.sync_copy(attl_hbm, albuf)

    def chunk(i, m):
        off = base + i * KC
        pltpu.sync_copy(src_hbm.at[pl.ds(off, KC)], srcb)
        pltpu.sync_copy(dst_hbm.at[pl.ds(off, KC)], dstb)
        pltpu.sync_copy(eaw_hbm.at[pl.ds(off, KC), :], eab)
        pltpu.async_copy(xw1_hbm.at[srcb], g1b, sem).wait()

        def edge(e, carry):
            acc = jnp.zeros((16,), jnp.float32)
            for cb in range(16):
                z = g1b[e, pl.ds(cb * 16, 16)] + eab[e, pl.ds(cb * 16, 16)]
                lz = jnp.where(z >= 0, z, 0.01 * z)
                acc = acc + lz * albuf[pl.ds(cb * 16, 16)]
            pbuf[pl.ds(e * 16, 16)] = acc
            return carry

        lax.fori_loop(0, KC, edge, 0)

        lane = lax.iota(jnp.int32, 16)

        def grp(j, mm):
            idx0 = (j * 16 + lane) * 16
            ssum = plsc.load_gather(pbuf, [idx0])
            for cc in range(1, 16):
                ssum = ssum + plsc.load_gather(pbuf, [idx0 + cc])
            dstv = dstb[pl.ds(j * 16, 16)]
            rv = plsc.load_gather(rbuf, [dstv])
            sv = ssum + rv
            sv = jnp.where(sv >= 0, sv, 0.01 * sv)
            sbuf[pl.ds(j * 16, 16)] = sv
            return jnp.maximum(mm, sv)

        m = lax.fori_loop(0, KC // 16, grp, m)
        pltpu.sync_copy(sbuf, s_hbm.at[pl.ds(off, KC)])
        return m

    m = lax.fori_loop(0, EW // KC, chunk, jnp.full((16,), -3e38, jnp.float32))
    mbuf[...] = m
    pltpu.sync_copy(mbuf, mx_hbm.at[wid])


def _sc_score_e(eaw, xw1, src, dst, rr, attl):
    f = pl.kernel(
        _sc_score_e_kernel,
        out_type=[
            jax.ShapeDtypeStruct((E,), jnp.float32),
            jax.ShapeDtypeStruct((NW, 16), jnp.float32),
        ],
        mesh=_MESH,
        scratch_types=[
            pltpu.VMEM((N,), jnp.float32),
            pltpu.VMEM((H,), jnp.float32),
            pltpu.VMEM((KC,), jnp.int32),
            pltpu.VMEM((KC,), jnp.int32),
            pltpu.VMEM((KC, H), jnp.float32),
            pltpu.VMEM((KC, H), jnp.float32),
            pltpu.VMEM((KC * 16,), jnp.float32),
            pltpu.VMEM((KC,), jnp.float32),
            pltpu.VMEM((16,), jnp.float32),
            pltpu.SemaphoreType.DMA,
        ],
    )
    return f(eaw, xw1, src, dst, rr, attl)


def _sc_score_c_kernel(as_hbm, ad_hbm, src_hbm, dst_hbm, s_hbm, mx_hbm,
                       asb, adb, srcb, dstb, sbuf, mbuf):
    wid = _widx()
    base = wid * EW
    KS = 400
    pltpu.sync_copy(as_hbm, asb)
    pltpu.sync_copy(ad_hbm, adb)

    def chunk(i, m):
        off = base + i * KS
        pltpu.sync_copy(src_hbm.at[pl.ds(off, KS)], srcb)
        pltpu.sync_copy(dst_hbm.at[pl.ds(off, KS)], dstb)

        def grp(j, mm):
            srcv = srcb[pl.ds(j * 16, 16)]
            dstv = dstb[pl.ds(j * 16, 16)]
            sv = plsc.load_gather(asb, [srcv]) + plsc.load_gather(adb, [dstv])
            sv = jnp.where(sv >= 0, sv, 0.01 * sv)
            sbuf[pl.ds(j * 16, 16)] = sv
            return jnp.maximum(mm, sv)

        m = lax.fori_loop(0, KS // 16, grp, m)
        pltpu.sync_copy(sbuf, s_hbm.at[pl.ds(off, KS)])
        return m

    m = lax.fori_loop(0, EW // KS, chunk, jnp.full((16,), -3e38, jnp.float32))
    mbuf[...] = m
    pltpu.sync_copy(mbuf, mx_hbm.at[wid])


def _sc_score_c(asrc, adst, src, dst):
    KS = 400
    f = pl.kernel(
        _sc_score_c_kernel,
        out_type=[
            jax.ShapeDtypeStruct((E,), jnp.float32),
            jax.ShapeDtypeStruct((NW, 16), jnp.float32),
        ],
        mesh=_MESH,
        scratch_types=[
            pltpu.VMEM((N,), jnp.float32),
            pltpu.VMEM((N,), jnp.float32),
            pltpu.VMEM((KS,), jnp.int32),
            pltpu.VMEM((KS,), jnp.int32),
            pltpu.VMEM((KS,), jnp.float32),
            pltpu.VMEM((16,), jnp.float32),
        ],
    )
    return f(asrc, adst, src, dst)


def _sc_agg_kernel(xw_hbm, src_hbm, dst_hbm, s_hbm, mx_hbm,
                   rows_hbm, d_hbm,
                   rows_sh, dsh, srcb, dstb, sbuf, idxb, rbuf, dbuf, ebuf,
                   mxb, sem, sem2):
    c = lax.axis_index("c")
    sid = lax.axis_index("s")
    lane = lax.iota(jnp.int32, 16)

    def zr(t, carry):
        rbuf[t // 16, pl.ds((t % 16) * 16, 16)] = jnp.zeros((16,), jnp.float32)
        return carry
    lax.fori_loop(0, KC * 16, zr, 0)

    def zd(t, carry):
        dbuf[t, :] = jnp.zeros((16,), jnp.float32)
        return carry
    lax.fori_loop(0, KC, zd, 0)

    rpt = NPAD // 16
    for r0 in (0, KC, 2 * KC, rpt - KC):
        pltpu.sync_copy(rbuf, rows_sh.at[pl.ds(sid * rpt + r0, KC), :])
        pltpu.sync_copy(dbuf, dsh.at[pl.ds(sid * rpt + r0, KC), :])
    plsc.subcore_barrier()

    pltpu.sync_copy(mx_hbm, mxb)

    def mred(i, m):
        return jnp.maximum(m, mxb[pl.ds(i * 16, 16)])
    m = lax.fori_loop(0, NW, mred, jnp.full((16,), -3e38, jnp.float32))
    G = jnp.max(m)

    nbase = c * NHALF

    def chunk(i, carry):
        off = sid * ET + i * KC
        pltpu.sync_copy(src_hbm.at[pl.ds(off, KC)], srcb)
        pltpu.sync_copy(dst_hbm.at[pl.ds(off, KC)], dstb)
        pltpu.sync_copy(s_hbm.at[pl.ds(off, KC)], sbuf)
        pltpu.async_copy(xw_hbm.at[srcb], rbuf, sem).wait()

        for j in range(KC // 16):
            sv = sbuf[pl.ds(j * 16, 16)]
            ev = jnp.exp(sv - G)
            ebuf[pl.ds(j * 16, 16)] = ev
            dstv = dstb[pl.ds(j * 16, 16)]
            loc = dstv - nbase
            ok = (loc >= 0) & (loc < NHALF)
            lidx = jnp.where(ok, loc, NHALF)
            idxb[pl.ds(j * 16, 16)] = lidx
            plsc.store_scatter(dbuf, [j * 16 + lane,
                                      jnp.zeros((16,), jnp.int32)], ev)

        def scale(e, carry2):
            be = plsc.load_gather(ebuf, [jnp.full((16,), e, jnp.int32)])
            for cb in range(16):
                v = rbuf[e, pl.ds(cb * 16, 16)]
                rbuf[e, pl.ds(cb * 16, 16)] = v * be
            return carry2
        lax.fori_loop(0, KC, scale, 0)

        pltpu.async_copy(rbuf, rows_sh.at[idxb], sem, add=True).wait()
        pltpu.async_copy(dbuf, dsh.at[idxb], sem2, add=True).wait()
        return carry

    lax.fori_loop(0, ET // KC, chunk, 0)
    plsc.subcore_barrier()

    for r0 in (0, KC, 2 * KC, rpt - KC):
        pltpu.sync_copy(rows_sh.at[pl.ds(sid * rpt + r0, KC), :],
                        rows_hbm.at[c, pl.ds(sid * rpt + r0, KC), :])
        pltpu.sync_copy(dsh.at[pl.ds(sid * rpt + r0, KC), :],
                        d_hbm.at[c, pl.ds(sid * rpt + r0, KC), :])


def _sc_agg(xw, src, dst, s, mx):
    f = pl.kernel(
        _sc_agg_kernel,
        out_type=[
            jax.ShapeDtypeStruct((2, NPAD, H), jnp.float32),
            jax.ShapeDtypeStruct((2, NPAD, 16), jnp.float32),
        ],
        mesh=_MESH,
        scratch_types=[
            pltpu.VMEM_SHARED((NPAD, H), jnp.float32),
            pltpu.VMEM_SHARED((NPAD, 16), jnp.float32),
            pltpu.VMEM((KC,), jnp.int32),
            pltpu.VMEM((KC,), jnp.int32),
            pltpu.VMEM((KC,), jnp.float32),
            pltpu.VMEM((KC,), jnp.int32),
            pltpu.VMEM((KC, H), jnp.float32),
            pltpu.VMEM((KC, 16), jnp.float32),
            pltpu.VMEM((KC,), jnp.float32),
            pltpu.VMEM((NW * 16,), jnp.float32),
            pltpu.SemaphoreType.DMA,
            pltpu.SemaphoreType.DMA,
        ],
    )
    return f(xw, src, dst, s, mx.reshape(NW * 16))


def kernel(x, edge_index, edge_attr, batch, params):
    p = params
    src = edge_index[0].astype(jnp.int32)
    dst = edge_index[1].astype(jnp.int32)
    batch2 = batch.astype(jnp.int32).reshape(N, 1)

    wl = p['g_lin1_W'][:, :H]
    wr = p['g_lin1_W'][:, H:]

    xemb, xw1, xw2, rr = _tc_a(x, p['lin1_W'], p['lin1_b'], wl,
                               p['g_lin2_W'], p['g_att_r'][0])
    eaw = _tc_eaw(edge_attr, wr)
    s, mx = _sc_score_e(eaw, xw1, src, dst, rr.reshape(N), p['g_att_l'][0])
    rows2, dd2 = _sc_agg(xw2, src, dst, s, mx)
    rows = jnp.concatenate([rows2[0, :NHALF], rows2[1, :NHALF]], 0)
    d = jnp.concatenate([dd2[0, :NHALF, 0], dd2[1, :NHALF, 0]], 0).reshape(N, 1)

    nxt = [
        (p['conv0_W'], p['conv0_att_src'], p['conv0_att_dst']),
        (p['conv1_W'], p['conv1_att_src'], p['conv1_att_dst']),
        (p['conv2_W'], p['conv2_att_src'], p['conv2_att_dst']),
        (p['mol_W'], p['mol_att_src'], p['mol_att_dst']),
    ]
    xcur, xl, asrc, adst = _tc_layer(
        rows, d, p['g_bias'], xemb, p['gru0_Wih'], p['gru0_Whh'],
        p['gru0_bih'], p['gru0_bhh'], *nxt[0])

    for l in range(3):
        s, mx = _sc_score_c(asrc.reshape(N), adst.reshape(N), src, dst)
        rows2, dd2 = _sc_agg(xl, src, dst, s, mx)
        rows = jnp.concatenate([rows2[0, :NHALF], rows2[1, :NHALF]], 0)
        d = jnp.concatenate([dd2[0, :NHALF, 0], dd2[1, :NHALF, 0]],
                            0).reshape(N, 1)
        g = 'gru%d' % (l + 1)
        xcur, xl, asrc, adst = _tc_layer(
            rows, d, p['conv%d_bias' % l], xcur, p[g + '_Wih'], p[g + '_Whh'],
            p[g + '_bih'], p[g + '_bhh'], *nxt[l + 1])

    out, cdst = _tc_r1(xcur, batch2, p['mol_W'], p['mol_att_dst'])
    for _ in range(2):
        s_n, g_max = _tc_r3(asrc, cdst, batch2)
        h = _tc_r4(xl, s_n, g_max, batch2, p['mol_bias'])
        out, cdst = _tc_r5(h, out, p['mol_gru_Wih'], p['mol_gru_Whh'],
                           p['mol_gru_bih'], p['mol_gru_bhh'],
                           p['mol_W'], p['mol_att_dst'])
    return _tc_r6(out, p['lin2_W'], p['lin2_b'])


# speedup vs baseline: 1.2803x; 1.2803x over previous
import functools
import jax
import jax.numpy as jnp
from jax import lax
from jax.experimental import pallas as pl
from jax.experimental.pallas import tpu as pltpu
from jax.experimental.pallas import tpu_sc as plsc

N = 10000
E = 320000
B = 64
H = 256
NW = 32
EW = E // NW
KC = 80
RPT = 313
TR = 313
ACCR = 320
SCK = 1280
GB = 64
LCAP = SCK + GB + 16

_MESH = plsc.VectorSubcoreMesh(core_axis_name="c", subcore_axis_name="s")


def _lrelu(v):
    return jnp.where(v >= 0, v, 0.01 * v)


def _dot_t(a, b):
    return lax.dot_general(a, b, (((1,), (1,)), ((), ())),
                           preferred_element_type=jnp.float32)


def _tc_a_body(x_ref, w1_ref, b1_ref, wl_ref, w2_ref, ar_ref,
               xe_ref, xw1_ref, xw2_ref, r_ref):
    xe = _lrelu(_dot_t(x_ref[...], w1_ref[...]) + b1_ref[...])
    xe_ref[...] = xe
    xw1_ref[...] = _dot_t(xe, wl_ref[...])
    xw2_ref[...] = _dot_t(xe, w2_ref[...])
    r_ref[...] = jnp.dot(xe, ar_ref[...], preferred_element_type=jnp.float32)


def _tc_a(x, w1, b1, wl, w2, ar):
    blk = 400
    f = pl.pallas_call(
        _tc_a_body,
        grid=(N // blk,),
        in_specs=[
            pl.BlockSpec((blk, 128), lambda i: (i, 0)),
            pl.BlockSpec((H, 128), lambda i: (0, 0)),
            pl.BlockSpec((1, H), lambda i: (0, 0)),
            pl.BlockSpec((H, H), lambda i: (0, 0)),
            pl.BlockSpec((H, H), lambda i: (0, 0)),
            pl.BlockSpec((H, 1), lambda i: (0, 0)),
        ],
        out_specs=[
            pl.BlockSpec((blk, H), lambda i: (i, 0)),
            pl.BlockSpec((blk, H), lambda i: (i, 0)),
            pl.BlockSpec((blk, H), lambda i: (i, 0)),
            pl.BlockSpec((blk, 1), lambda i: (i, 0)),
        ],
        out_shape=[
            jax.ShapeDtypeStruct((N, H), jnp.float32),
            jax.ShapeDtypeStruct((N, H), jnp.float32),
            jax.ShapeDtypeStruct((N, H), jnp.float32),
            jax.ShapeDtypeStruct((N, 1), jnp.float32),
        ],
    )
    return f(x, w1, b1.reshape(1, H), wl, w2, ar.reshape(H, 1))


def _tc_eaw_body(ea_ref, wrt_ref, out_ref):
    out_ref[...] = jnp.dot(ea_ref[...], wrt_ref[...],
                           preferred_element_type=jnp.float32)


def _tc_eaw(ea, wr):
    blk = 1600
    f = pl.pallas_call(
        _tc_eaw_body,
        grid=(E // blk,),
        in_specs=[
            pl.BlockSpec((blk, 16), lambda i: (i, 0)),
            pl.BlockSpec((16, H), lambda i: (0, 0)),
        ],
        out_specs=pl.BlockSpec((blk, H), lambda i: (i, 0)),
        out_shape=jax.ShapeDtypeStruct((E, H), jnp.float32),
    )
    return f(ea, wr.T)


def _tc_layer_body(rows_ref, d_ref, ba_ref, x_ref, wih_ref, whh_ref,
                   bih_ref, bhh_ref, wn_ref, ans_ref, and_ref,
                   xn_ref, xl_ref, as_ref, ad_ref):
    h = rows_ref[...] / (d_ref[...] + 1e-16) + ba_ref[...]
    h = jnp.where(h > 0, h, jnp.exp(h) - 1.0)
    xold = x_ref[...]
    gi = _dot_t(h, wih_ref[...]) + bih_ref[...]
    gh = _dot_t(xold, whh_ref[...]) + bhh_ref[...]
    r = jax.nn.sigmoid(gi[:, :H] + gh[:, :H])
    z = jax.nn.sigmoid(gi[:, H:2 * H] + gh[:, H:2 * H])
    n = jnp.tanh(gi[:, 2 * H:] + r * gh[:, 2 * H:])
    xn = jnp.maximum((1.0 - z) * n + z * xold, 0.0)
    xn_ref[...] = xn
    xl = _dot_t(xn, wn_ref[...])
    xl_ref[...] = xl
    as_ref[...] = jnp.dot(xl, ans_ref[...], preferred_element_type=jnp.float32)
    ad_ref[...] = jnp.dot(xl, and_ref[...], preferred_element_type=jnp.float32)


def _tc_layer(rows, d, ba, x, wih, whh, bih, bhh, wn, ans, andst):
    blk = 400
    f = pl.pallas_call(
        _tc_layer_body,
        grid=(N // blk,),
        in_specs=[
            pl.BlockSpec((blk, H), lambda i: (i, 0)),
            pl.BlockSpec((blk, 1), lambda i: (i, 0)),
            pl.BlockSpec((1, H), lambda i: (0, 0)),
            pl.BlockSpec((blk, H), lambda i: (i, 0)),
            pl.BlockSpec((3 * H, H), lambda i: (0, 0)),
            pl.BlockSpec((3 * H, H), lambda i: (0, 0)),
            pl.BlockSpec((1, 3 * H), lambda i: (0, 0)),
            pl.BlockSpec((1, 3 * H), lambda i: (0, 0)),
            pl.BlockSpec((H, H), lambda i: (0, 0)),
            pl.BlockSpec((H, 1), lambda i: (0, 0)),
            pl.BlockSpec((H, 1), lambda i: (0, 0)),
        ],
        out_specs=[
            pl.BlockSpec((blk, H), lambda i: (i, 0)),
            pl.BlockSpec((blk, H), lambda i: (i, 0)),
            pl.BlockSpec((blk, 1), lambda i: (i, 0)),
            pl.BlockSpec((blk, 1), lambda i: (i, 0)),
        ],
        out_shape=[
            jax.ShapeDtypeStruct((N, H), jnp.float32),
            jax.ShapeDtypeStruct((N, H), jnp.float32),
            jax.ShapeDtypeStruct((N, 1), jnp.float32),
            jax.ShapeDtypeStruct((N, 1), jnp.float32),
        ],
    )
    return f(rows, d, ba.reshape(1, H), x, wih, whh, bih.reshape(1, 3 * H),
             bhh.reshape(1, 3 * H), wn, ans.reshape(H, 1), andst.reshape(H, 1))


def _mask_t(batch_blk):
    blk = batch_blk.shape[0]
    cols = lax.broadcasted_iota(jnp.int32, (blk, B), 1)
    return jnp.where(batch_blk == cols, 1.0, 0.0).astype(jnp.float32)


def _dot_c0(a, b):
    return lax.dot_general(a, b, (((0,), (0,)), ((), ())),
                           preferred_element_type=jnp.float32)


def _tc_r1_body(x_ref, b_ref, molw_ref, ad_ref, out_ref, cd_ref, acc):
    i = pl.program_id(0)

    @pl.when(i == 0)
    def _():
        acc[...] = jnp.zeros_like(acc)

    mt = _mask_t(b_ref[...])
    acc[...] += _dot_c0(mt, x_ref[...])

    @pl.when(i == pl.num_programs(0) - 1)
    def _():
        out0 = jnp.maximum(acc[...], 0.0)
        out_ref[...] = out0
        od = _dot_t(out0, molw_ref[...])
        cd_ref[...] = jnp.dot(od, ad_ref[...], preferred_element_type=jnp.float32)


def _tc_r1(x, batch, molw, attdst):
    blk = 400
    f = pl.pallas_call(
        _tc_r1_body,
        grid=(N // blk,),
        in_specs=[
            pl.BlockSpec((blk, H), lambda i: (i, 0)),
            pl.BlockSpec((blk, 1), lambda i: (i, 0)),
            pl.BlockSpec((H, H), lambda i: (0, 0)),
            pl.BlockSpec((H, 1), lambda i: (0, 0)),
        ],
        out_specs=[
            pl.BlockSpec((B, H), lambda i: (0, 0)),
            pl.BlockSpec((B, 1), lambda i: (0, 0)),
        ],
        out_shape=[
            jax.ShapeDtypeStruct((B, H), jnp.float32),
            jax.ShapeDtypeStruct((B, 1), jnp.float32),
        ],
        scratch_shapes=[pltpu.VMEM((B, H), jnp.float32)],
    )
    return f(x, batch, molw, attdst.reshape(H, 1))


def _tc_r3_body(cs_ref, cd_ref, b_ref, s_ref, g_ref, gacc):
    i = pl.program_id(0)
    mt = _mask_t(b_ref[...])
    s = _lrelu(cs_ref[...] + jnp.dot(mt, cd_ref[...],
                                     preferred_element_type=jnp.float32))
    s_ref[...] = s
    m = jnp.max(s)

    @pl.when(i == 0)
    def _():
        gacc[0, 0] = m

    @pl.when(i > 0)
    def _():
        gacc[0, 0] = jnp.maximum(gacc[0, 0], m)

    @pl.when(i == pl.num_programs(0) - 1)
    def _():
        g_ref[...] = jnp.full((1, 1), gacc[0, 0], jnp.float32)


def _tc_r3(csrc, cdst, batch):
    blk = 400
    f = pl.pallas_call(
        _tc_r3_body,
        grid=(N // blk,),
        in_specs=[
            pl.BlockSpec((blk, 1), lambda i: (i, 0)),
            pl.BlockSpec((B, 1), lambda i: (0, 0)),
            pl.BlockSpec((blk, 1), lambda i: (i, 0)),
        ],
        out_specs=[
            pl.BlockSpec((blk, 1), lambda i: (i, 0)),
            pl.BlockSpec((1, 1), lambda i: (0, 0)),
        ],
        out_shape=[
            jax.ShapeDtypeStruct((N, 1), jnp.float32),
            jax.ShapeDtypeStruct((1, 1), jnp.float32),
        ],
        scratch_shapes=[pltpu.SMEM((1, 1), jnp.float32)],
    )
    return f(csrc, cdst, batch)


def _tc_r4_body(xs_ref, s_ref, g_ref, b_ref, bias_ref, h_ref, dacc, hacc):
    i = pl.program_id(0)

    @pl.when(i == 0)
    def _():
        dacc[...] = jnp.zeros_like(dacc)
        hacc[...] = jnp.zeros_like(hacc)

    mt = _mask_t(b_ref[...])
    e = jnp.exp(s_ref[...] - g_ref[...])
    dacc[...] += _dot_c0(mt, e)
    hacc[...] += _dot_c0(mt, xs_ref[...] * e)

    @pl.when(i == pl.num_programs(0) - 1)
    def _():
        h = hacc[...] / (dacc[...] + 1e-16) + bias_ref[...]
        h_ref[...] = jnp.where(h > 0, h, jnp.exp(h) - 1.0)


def _tc_r4(xs, s, g, batch, bias):
    blk = 400
    f = pl.pallas_call(
        _tc_r4_body,
        grid=(N // blk,),
        in_specs=[
            pl.BlockSpec((blk, H), lambda i: (i, 0)),
            pl.BlockSpec((blk, 1), lambda i: (i, 0)),
            pl.BlockSpec((1, 1), lambda i: (0, 0)),
            pl.BlockSpec((blk, 1), lambda i: (i, 0)),
            pl.BlockSpec((1, H), lambda i: (0, 0)),
        ],
        out_specs=pl.BlockSpec((B, H), lambda i: (0, 0)),
        out_shape=jax.ShapeDtypeStruct((B, H), jnp.float32),
        scratch_shapes=[pltpu.VMEM((B, 1), jnp.float32),
                        pltpu.VMEM((B, H), jnp.float32)],
    )
    return f(xs, s, g, batch, bias.reshape(1, H))


def _tc_r5_body(h_ref, o_ref, wih_ref, whh_ref, bih_ref, bhh_ref,
                molw_ref, ad_ref, on_ref, cd_ref):
    h = h_ref[...]
    oo = o_ref[...]
    gi = _dot_t(h, wih_ref[...]) + bih_ref[...]
    gh = _dot_t(oo, whh_ref[...]) + bhh_ref[...]
    r = jax.nn.sigmoid(gi[:, :H] + gh[:, :H])
    z = jax.nn.sigmoid(gi[:, H:2 * H] + gh[:, H:2 * H])
    n = jnp.tanh(gi[:, 2 * H:] + r * gh[:, 2 * H:])
    on = jnp.maximum((1.0 - z) * n + z * oo, 0.0)
    on_ref[...] = on
    od = _dot_t(on, molw_ref[...])
    cd_ref[...] = jnp.dot(od, ad_ref[...], preferred_element_type=jnp.float32)


def _tc_r5(h, out, wih, whh, bih, bhh, molw, attdst):
    f = pl.pallas_call(
        _tc_r5_body,
        out_shape=[
            jax.ShapeDtypeStruct((B, H), jnp.float32),
            jax.ShapeDtypeStruct((B, 1), jnp.float32),
        ],
    )
    return f(h, out, wih, whh, bih.reshape(1, 3 * H), bhh.reshape(1, 3 * H),
             molw, attdst.reshape(H, 1))


def _tc_r6_body(o_ref, w_ref, b_ref, out_ref):
    out_ref[...] = _dot_t(o_ref[...], w_ref[...]) + b_ref[...]


def _tc_r6(out, w, b):
    f = pl.pallas_call(
        _tc_r6_body,
        out_shape=jax.ShapeDtypeStruct((B, w.shape[0]), jnp.float32),
    )
    return f(out, w, b.reshape(1, -1))


def _widx():
    return lax.axis_index("c") * 16 + lax.axis_index("s")


def _sc_score_e_kernel(eaw_hbm, xw1_hbm, src_hbm, dst_hbm, rr_hbm, attl_hbm,
                       s_hbm, mx_hbm,
                       rbuf, albuf, srcb, dstb, eab, g1b, pbuf, sbuf, mbuf,
                       sem):
    wid = _widx()
    base = wid * EW
    pltpu.sync_copy(rr_hbm, rbuf)
    pltpu.sync_copy(attl_hbm, albuf)

    def chunk(i, m):
        off = base + i * KC
        pltpu.sync_copy(src_hbm.at[pl.ds(off, KC)], srcb)
        pltpu.sync_copy(dst_hbm.at[pl.ds(off, KC)], dstb)
        pltpu.sync_copy(eaw_hbm.at[pl.ds(off, KC), :], eab)
        pltpu.async_copy(xw1_hbm.at[srcb], g1b, sem).wait()

        def edge(e, carry):
            acc = jnp.zeros((16,), jnp.float32)
            for cb in range(16):
                z = g1b[e, pl.ds(cb * 16, 16)] + eab[e, pl.ds(cb * 16, 16)]
                lz = jnp.where(z >= 0, z, 0.01 * z)
                acc = acc + lz * albuf[pl.ds(cb * 16, 16)]
            pbuf[pl.ds(e * 16, 16)] = acc
            return carry

        lax.fori_loop(0, KC, edge, 0)

        lane = lax.iota(jnp.int32, 16)

        def grp(j, mm):
            idx0 = (j * 16 + lane) * 16
            ssum = plsc.load_gather(pbuf, [idx0])
            for cc in range(1, 16):
                ssum = ssum + plsc.load_gather(pbuf, [idx0 + cc])
            dstv = dstb[pl.ds(j * 16, 16)]
            rv = plsc.load_gather(rbuf, [dstv])
            sv = ssum + rv
            sv = jnp.where(sv >= 0, sv, 0.01 * sv)
            sbuf[pl.ds(j * 16, 16)] = sv
            return jnp.maximum(mm, sv)

        m = lax.fori_loop(0, KC // 16, grp, m)
        pltpu.sync_copy(sbuf, s_hbm.at[pl.ds(off, KC)])
        return m

    m = lax.fori_loop(0, EW // KC, chunk, jnp.full((16,), -3e38, jnp.float32))
    mbuf[...] = m
    pltpu.sync_copy(mbuf, mx_hbm.at[wid])


def _sc_score_e(eaw, xw1, src, dst, rr, attl):
    f = pl.kernel(
        _sc_score_e_kernel,
        out_type=[
            jax.ShapeDtypeStruct((E,), jnp.float32),
            jax.ShapeDtypeStruct((NW, 16), jnp.float32),
        ],
        mesh=_MESH,
        compiler_params=pltpu.CompilerParams(needs_layout_passes=False),
        scratch_types=[
            pltpu.VMEM((N,), jnp.float32),
            pltpu.VMEM((H,), jnp.float32),
            pltpu.VMEM((KC,), jnp.int32),
            pltpu.VMEM((KC,), jnp.int32),
            pltpu.VMEM((KC, H), jnp.float32),
            pltpu.VMEM((KC, H), jnp.float32),
            pltpu.VMEM((KC * 16,), jnp.float32),
            pltpu.VMEM((KC,), jnp.float32),
            pltpu.VMEM((16,), jnp.float32),
            pltpu.SemaphoreType.DMA,
        ],
    )
    return f(eaw, xw1, src, dst, rr, attl)


def _sc_score_c_kernel(as_hbm, ad_hbm, src_hbm, dst_hbm, s_hbm, mx_hbm,
                       asb, adb, srcb, dstb, sbuf, mbuf):
    wid = _widx()
    base = wid * EW
    KS = 400
    pltpu.sync_copy(as_hbm, asb)
    pltpu.sync_copy(ad_hbm, adb)

    def chunk(i, m):
        off = base + i * KS
        pltpu.sync_copy(src_hbm.at[pl.ds(off, KS)], srcb)
        pltpu.sync_copy(dst_hbm.at[pl.ds(off, KS)], dstb)

        def grp(j, mm):
            srcv = srcb[pl.ds(j * 16, 16)]
            dstv = dstb[pl.ds(j * 16, 16)]
            sv = plsc.load_gather(asb, [srcv]) + plsc.load_gather(adb, [dstv])
            sv = jnp.where(sv >= 0, sv, 0.01 * sv)
            sbuf[pl.ds(j * 16, 16)] = sv
            return jnp.maximum(mm, sv)

        m = lax.fori_loop(0, KS // 16, grp, m)
        pltpu.sync_copy(sbuf, s_hbm.at[pl.ds(off, KS)])
        return m

    m = lax.fori_loop(0, EW // KS, chunk, jnp.full((16,), -3e38, jnp.float32))
    mbuf[...] = m
    pltpu.sync_copy(mbuf, mx_hbm.at[wid])


def _sc_score_c(asrc, adst, src, dst):
    KS = 400
    f = pl.kernel(
        _sc_score_c_kernel,
        out_type=[
            jax.ShapeDtypeStruct((E,), jnp.float32),
            jax.ShapeDtypeStruct((NW, 16), jnp.float32),
        ],
        mesh=_MESH,
        compiler_params=pltpu.CompilerParams(needs_layout_passes=False),
        scratch_types=[
            pltpu.VMEM((N,), jnp.float32),
            pltpu.VMEM((N,), jnp.float32),
            pltpu.VMEM((KS,), jnp.int32),
            pltpu.VMEM((KS,), jnp.int32),
            pltpu.VMEM((KS,), jnp.float32),
            pltpu.VMEM((16,), jnp.float32),
        ],
    )
    return f(asrc, adst, src, dst)


def _sc_agg_kernel(xw_hbm, src_hbm, dst_hbm, s_hbm, mx_hbm,
                   rows_hbm, d_hbm,
                   acc, dacc, srcb, dstb, sb, lsrc, lloc, lwt, gbuf, mxb,
                   sem):
    w = _widx()
    base = w * RPT
    lane = lax.iota(jnp.int32, 16)

    def zr(t, carry):
        acc[t // 16, pl.ds((t % 16) * 16, 16)] = jnp.zeros((16,), jnp.float32)
        return carry
    lax.fori_loop(0, ACCR * 16, zr, 0)

    def zd(t, carry):
        dacc[pl.ds(t * 16, 16)] = jnp.zeros((16,), jnp.float32)
        return carry
    lax.fori_loop(0, ACCR, zd, 0)

    pltpu.sync_copy(mx_hbm, mxb)

    def mred(i, m):
        return jnp.maximum(m, mxb[pl.ds(i * 16, 16)])
    m = lax.fori_loop(0, NW, mred, jnp.full((16,), -3e38, jnp.float32))
    G = jnp.max(m)

    zf = jnp.zeros((16,), jnp.float32)
    zi = jnp.zeros((16,), jnp.int32)
    trv = jnp.full((16,), TR, jnp.int32)

    def chunk(i, cnt):
        off = i * SCK
        pltpu.sync_copy(src_hbm.at[pl.ds(off, SCK)], srcb)
        pltpu.sync_copy(dst_hbm.at[pl.ds(off, SCK)], dstb)
        pltpu.sync_copy(s_hbm.at[pl.ds(off, SCK)], sb)

        def grp(j, cn):
            dstv = dstb[pl.ds(j * 16, 16)]
            srcv = srcb[pl.ds(j * 16, 16)]
            sv = sb[pl.ds(j * 16, 16)]
            loc = dstv - base
            okm = (loc >= 0) & (loc < RPT)
            ev = jnp.exp(sv - G)
            pc = plsc.cumsum(okm.astype(jnp.int32))
            pos = cn + pc - 1
            plsc.store_scatter(lsrc, [pos], srcv, mask=okm)
            plsc.store_scatter(lloc, [pos], loc, mask=okm)
            plsc.store_scatter(lwt, [pos], ev, mask=okm)
            return cn + plsc.all_reduce_population_count(okm)[0]

        cnt = lax.fori_loop(0, SCK // 16, grp, cnt)

        nb = (cnt + GB - 1) // GB
        for g in range(GB // 16):
            pos = cnt + g * 16 + lane
            mm = pos < nb * GB
            plsc.store_scatter(lsrc, [pos], zi, mask=mm)
            plsc.store_scatter(lloc, [pos], trv, mask=mm)
            plsc.store_scatter(lwt, [pos], zf, mask=mm)

        def batch(b, carry):
            pltpu.async_copy(xw_hbm.at[lsrc.at[pl.ds(b * GB, GB)]], gbuf,
                             sem).wait()

            def srow(e2, carry2):
                posn = b * GB + e2
                loc = lloc[pl.ds(posn, 16)][0]
                wv = plsc.load_gather(lwt, [jnp.full((16,), posn, jnp.int32)])
                for cb in range(16):
                    v = gbuf[e2, pl.ds(cb * 16, 16)] * wv
                    plsc.addupdate(acc.at[loc, pl.ds(cb * 16, 16)], v)
                plsc.addupdate(dacc.at[pl.ds(loc * 16, 16)],
                               jnp.where(lane == 0, wv, 0.0))
                return carry2

            lax.fori_loop(0, GB, srow, 0)
            return carry

        lax.fori_loop(0, nb, batch, 0)
        return jnp.zeros((), jnp.int32)

    lax.fori_loop(0, E // SCK, chunk, jnp.zeros((), jnp.int32))

    pltpu.sync_copy(acc, rows_hbm.at[w])
    pltpu.sync_copy(dacc, d_hbm.at[w])


def _sc_agg(xw, src, dst, s, mx):
    f = pl.kernel(
        _sc_agg_kernel,
        out_type=[
            jax.ShapeDtypeStruct((NW, ACCR, H), jnp.float32),
            jax.ShapeDtypeStruct((NW, ACCR * 16), jnp.float32),
        ],
        mesh=_MESH,
        compiler_params=pltpu.CompilerParams(needs_layout_passes=False),
        scratch_types=[
            pltpu.VMEM((ACCR, H), jnp.float32),
            pltpu.VMEM((ACCR * 16,), jnp.float32),
            pltpu.VMEM((SCK,), jnp.int32),
            pltpu.VMEM((SCK,), jnp.int32),
            pltpu.VMEM((SCK,), jnp.float32),
            pltpu.VMEM((LCAP,), jnp.int32),
            pltpu.VMEM((LCAP,), jnp.int32),
            pltpu.VMEM((LCAP,), jnp.float32),
            pltpu.VMEM((GB, H), jnp.float32),
            pltpu.VMEM((NW * 16,), jnp.float32),
            pltpu.SemaphoreType.DMA,
        ],
    )
    return f(xw, src, dst, s, mx.reshape(NW * 16))


def kernel(x, edge_index, edge_attr, batch, params):
    p = params
    src = edge_index[0].astype(jnp.int32)
    dst = edge_index[1].astype(jnp.int32)
    batch2 = batch.astype(jnp.int32).reshape(N, 1)

    wl = p['g_lin1_W'][:, :H]
    wr = p['g_lin1_W'][:, H:]

    xemb, xw1, xw2, rr = _tc_a(x, p['lin1_W'], p['lin1_b'], wl,
                               p['g_lin2_W'], p['g_att_r'][0])
    eaw = _tc_eaw(edge_attr, wr)
    s, mx = _sc_score_e(eaw, xw1, src, dst, rr.reshape(N), p['g_att_l'][0])
    rows2, dd2 = _sc_agg(xw2, src, dst, s, mx)
    rows = rows2[:, :RPT, :].reshape(NW * RPT, H)[:N]
    d = dd2.reshape(NW, ACCR, 16)[:, :RPT, 0].reshape(NW * RPT)[:N].reshape(N, 1)

    nxt = [
        (p['conv0_W'], p['conv0_att_src'], p['conv0_att_dst']),
        (p['conv1_W'], p['conv1_att_src'], p['conv1_att_dst']),
        (p['conv2_W'], p['conv2_att_src'], p['conv2_att_dst']),
        (p['mol_W'], p['mol_att_src'], p['mol_att_dst']),
    ]
    xcur, xl, asrc, adst = _tc_layer(
        rows, d, p['g_bias'], xemb, p['gru0_Wih'], p['gru0_Whh'],
        p['gru0_bih'], p['gru0_bhh'], *nxt[0])

    for l in range(3):
        s, mx = _sc_score_c(asrc.reshape(N), adst.reshape(N), src, dst)
        rows2, dd2 = _sc_agg(xl, src, dst, s, mx)
        rows = rows2[:, :RPT, :].reshape(NW * RPT, H)[:N]
        d = dd2.reshape(NW, ACCR, 16)[:, :RPT, 0].reshape(NW * RPT)[:N].reshape(N, 1)
        g = 'gru%d' % (l + 1)
        xcur, xl, asrc, adst = _tc_layer(
            rows, d, p['conv%d_bias' % l], xcur, p[g + '_Wih'], p[g + '_Whh'],
            p[g + '_bih'], p[g + '_bhh'], *nxt[l + 1])

    out, cdst = _tc_r1(xcur, batch2, p['mol_W'], p['mol_att_dst'])
    for _ in range(2):
        s_n, g_max = _tc_r3(asrc, cdst, batch2)
        h = _tc_r4(xl, s_n, g_max, batch2, p['mol_bias'])
        out, cdst = _tc_r5(h, out, p['mol_gru_Wih'], p['mol_gru_Whh'],
                           p['mol_gru_bih'], p['mol_gru_bhh'],
                           p['mol_W'], p['mol_att_dst'])
    return _tc_r6(out, p['lin2_W'], p['lin2_b'])
